# baseline (speedup 1.0000x reference)
import jax
import jax.numpy as jnp
from jax.experimental import pallas as pl
from jax.experimental.pallas import tpu as pltpu

_TN = 256


def _chamfer_body(pred_ref, gt_ref, out_ref, colmin_ref, sums_ref):
    b = pl.program_id(0)
    n = pl.program_id(1)
    nb = pl.num_programs(0)
    nn = pl.num_programs(1)

    @pl.when(jnp.logical_and(b == 0, n == 0))
    def _init():
        sums_ref[0] = 0.0
        sums_ref[1] = 0.0

    p = pred_ref[0]
    g = gt_ref[0]
    p2 = jnp.sum(p * p, axis=1, keepdims=True)
    g2 = jnp.sum(g * g, axis=0, keepdims=True)
    cross = jax.lax.dot_general(
        p.astype(jnp.bfloat16), g.astype(jnp.bfloat16),
        (((1,), (0,)), ((), ())),
        preferred_element_type=jnp.float32,
    )
    d = p2 + g2 - 2.0 * cross

    sums_ref[0] += jnp.sum(jnp.min(d, axis=1))
    tile_colmin = jnp.min(d, axis=0, keepdims=True)

    @pl.when(n == 0)
    def _first():
        colmin_ref[...] = tile_colmin

    @pl.when(n > 0)
    def _rest():
        colmin_ref[...] = jnp.minimum(colmin_ref[...], tile_colmin)

    @pl.when(n == nn - 1)
    def _batch_done():
        sums_ref[1] += jnp.sum(colmin_ref[...])

    @pl.when(jnp.logical_and(b == nb - 1, n == nn - 1))
    def _finish():
        inv_n = 1.0 / (nb * nn * _TN)
        inv_m = 1.0 / (nb * colmin_ref.shape[1])
        loss = sums_ref[0] * inv_n + sums_ref[1] * inv_m
        out_ref[...] = jnp.reshape(loss, (1, 1))


def kernel(prediction, gt):
    bsz, n, _ = prediction.shape
    m = gt.shape[1]
    gt_t = jnp.transpose(gt, (0, 2, 1))

    out = pl.pallas_call(
        _chamfer_body,
        grid=(bsz, n // _TN),
        in_specs=[
            pl.BlockSpec((1, _TN, 3), lambda b, i: (b, i, 0)),
            pl.BlockSpec((1, 3, m), lambda b, i: (b, 0, 0)),
        ],
        out_specs=pl.BlockSpec((1, 1), lambda b, i: (0, 0)),
        out_shape=jax.ShapeDtypeStruct((1, 1), jnp.float32),
        scratch_shapes=[
            pltpu.VMEM((1, m), jnp.float32),
            pltpu.SMEM((2,), jnp.float32),
        ],
    )(prediction, gt_t)
    return out[0, 0]
